# baseline (speedup 1.0000x reference)
import jax
import jax.numpy as jnp
from jax.experimental import pallas as pl
from jax.experimental.pallas import tpu as pltpu

NUM_EXPERTS = 64
TOP_K = 8
D_MODEL = 4096
TOKENS = 16384

T_BLK = 512


def _router_block(x_ref, wt_ref, b_ref, w_out, i_out, psum_out, zsum_out):
    logits = jnp.dot(
        x_ref[...], wt_ref[...], preferred_element_type=jnp.float32
    )

    lt = logits.T + b_ref[...]

    zsum_out[0, 0, :] = jnp.sum(lt * lt, axis=1)

    m = jnp.max(lt, axis=0, keepdims=True)
    e = jnp.exp(lt - m)
    s = jnp.sum(e, axis=0, keepdims=True)
    probs = e / s

    psum_out[0, 0, :] = jnp.sum(probs, axis=1)

    sub = jax.lax.broadcasted_iota(jnp.int32, probs.shape, 0)
    vals = probs
    ws = []
    idxs = []
    for _ in range(TOP_K):
        mk = jnp.max(vals, axis=0, keepdims=True)
        is_mk = vals >= mk
        idx = jnp.min(
            jnp.where(is_mk, sub, NUM_EXPERTS), axis=0, keepdims=True
        )
        ws.append(mk)
        idxs.append(idx)
        vals = jnp.where(sub == idx, -1.0, vals)

    w_cat = jnp.concatenate(ws, axis=0)
    wsum = jnp.sum(w_cat, axis=0, keepdims=True)
    w_out[...] = (w_cat / (wsum + 1e-8)).T
    i_out[...] = jnp.concatenate(idxs, axis=0).T


@jax.jit
def kernel(x, W, expert_bias):
    grid = TOKENS // T_BLK
    w_t = W.T
    bias = expert_bias.reshape(NUM_EXPERTS, 1)

    w_out, i_out, psum, zsum = pl.pallas_call(
        _router_block,
        grid=(grid,),
        in_specs=[
            pl.BlockSpec((T_BLK, D_MODEL), lambda i: (i, 0)),
            pl.BlockSpec((D_MODEL, NUM_EXPERTS), lambda i: (0, 0)),
            pl.BlockSpec((NUM_EXPERTS, 1), lambda i: (0, 0)),
        ],
        out_specs=[
            pl.BlockSpec((T_BLK, TOP_K), lambda i: (i, 0)),
            pl.BlockSpec((T_BLK, TOP_K), lambda i: (i, 0)),
            pl.BlockSpec((1, 1, NUM_EXPERTS), lambda i: (i, 0, 0)),
            pl.BlockSpec((1, 1, NUM_EXPERTS), lambda i: (i, 0, 0)),
        ],
        out_shape=[
            jax.ShapeDtypeStruct((TOKENS, TOP_K), jnp.float32),
            jax.ShapeDtypeStruct((TOKENS, TOP_K), jnp.int32),
            jax.ShapeDtypeStruct((grid, 1, NUM_EXPERTS), jnp.float32),
            jax.ShapeDtypeStruct((grid, 1, NUM_EXPERTS), jnp.float32),
        ],
        compiler_params=pltpu.CompilerParams(
            dimension_semantics=("parallel",),
        ),
    )(x, w_t, bias)

    tokens_per_expert = jnp.sum(psum, axis=(0, 1)) / TOKENS
    uniform = 1.0 / NUM_EXPERTS
    load_balance_loss = (
        jnp.sum((tokens_per_expert - uniform) ** 2) * NUM_EXPERTS
    )
    router_z_loss = jnp.sum(zsum) / (TOKENS * NUM_EXPERTS) * 0.001
    return (w_out, i_out, load_balance_loss, router_z_loss)


# speedup vs baseline: 1.1363x; 1.1363x over previous
import jax
import jax.numpy as jnp
from jax.experimental import pallas as pl
from jax.experimental.pallas import tpu as pltpu

NUM_EXPERTS = 64
TOP_K = 8
D_MODEL = 4096
TOKENS = 16384

T_BLK = 1024


def _router_block(x_ref, w_ref, b_ref, w_out, i_out, lbl_out, zl_out,
                  psum_acc, zsum_acc):
    step = pl.program_id(0)
    nsteps = pl.num_programs(0)

    logits = jax.lax.dot_general(
        x_ref[...], w_ref[...],
        dimension_numbers=(((1,), (1,)), ((), ())),
        preferred_element_type=jnp.float32,
    )

    lt = logits.T + b_ref[...]

    zpart = jnp.sum(lt * lt, axis=1, keepdims=True)

    m = jnp.max(lt, axis=0, keepdims=True)
    e = jnp.exp(lt - m)
    s = jnp.sum(e, axis=0, keepdims=True)
    probs = e / s

    ppart = jnp.sum(probs, axis=1, keepdims=True)

    @pl.when(step == 0)
    def _init():
        psum_acc[...] = ppart
        zsum_acc[...] = zpart

    @pl.when(step != 0)
    def _accum():
        psum_acc[...] += ppart
        zsum_acc[...] += zpart

    @pl.when(step == nsteps - 1)
    def _finalize():
        tpe = psum_acc[...] / TOKENS
        u = 1.0 / NUM_EXPERTS
        lbl_out[0, 0] = jnp.sum((tpe - u) ** 2) * NUM_EXPERTS
        zl_out[0, 0] = jnp.sum(zsum_acc[...]) / (TOKENS * NUM_EXPERTS) * 0.001

    sub = jax.lax.broadcasted_iota(jnp.int32, probs.shape, 0)
    vals = probs
    ws = []
    idxs = []
    for _ in range(TOP_K):
        mk = jnp.max(vals, axis=0, keepdims=True)
        is_mk = vals >= mk
        idx = jnp.min(
            jnp.where(is_mk, sub, NUM_EXPERTS), axis=0, keepdims=True
        )
        ws.append(mk)
        idxs.append(idx)
        vals = jnp.where(sub == idx, -1.0, vals)

    w_cat = jnp.concatenate(ws, axis=0)
    wsum = jnp.sum(w_cat, axis=0, keepdims=True)
    w_out[...] = (w_cat / (wsum + 1e-8)).T
    i_out[...] = jnp.concatenate(idxs, axis=0).T


@jax.jit
def kernel(x, W, expert_bias):
    grid = TOKENS // T_BLK
    bias = expert_bias.reshape(NUM_EXPERTS, 1)

    w_out, i_out, lbl, zl = pl.pallas_call(
        _router_block,
        grid=(grid,),
        in_specs=[
            pl.BlockSpec((T_BLK, D_MODEL), lambda i: (i, 0)),
            pl.BlockSpec((NUM_EXPERTS, D_MODEL), lambda i: (0, 0)),
            pl.BlockSpec((NUM_EXPERTS, 1), lambda i: (0, 0)),
        ],
        out_specs=[
            pl.BlockSpec((T_BLK, TOP_K), lambda i: (i, 0)),
            pl.BlockSpec((T_BLK, TOP_K), lambda i: (i, 0)),
            pl.BlockSpec(memory_space=pltpu.SMEM),
            pl.BlockSpec(memory_space=pltpu.SMEM),
        ],
        out_shape=[
            jax.ShapeDtypeStruct((TOKENS, TOP_K), jnp.float32),
            jax.ShapeDtypeStruct((TOKENS, TOP_K), jnp.int32),
            jax.ShapeDtypeStruct((1, 1), jnp.float32),
            jax.ShapeDtypeStruct((1, 1), jnp.float32),
        ],
        scratch_shapes=[
            pltpu.VMEM((NUM_EXPERTS, 1), jnp.float32),
            pltpu.VMEM((NUM_EXPERTS, 1), jnp.float32),
        ],
        compiler_params=pltpu.CompilerParams(
            dimension_semantics=("arbitrary",),
        ),
    )(x, W, bias)

    return (w_out, i_out, lbl.reshape(()), zl.reshape(()))
